# baseline (speedup 1.0000x reference)
import jax
import jax.numpy as jnp
from jax.experimental import pallas as pl
from jax.experimental.pallas import tpu as pltpu

_E = 1600000
_GRID = 50
_IDX_ROWS = (_E * 2) // 128 // _GRID
_VAL_ROWS = _E // 128 // _GRID

_INT_MIN = jnp.iinfo(jnp.int32).min


def _fused_body(idx_ref, val_ref, idx_out, val_out, shape_out, acc_ref):
    i = pl.program_id(0)
    idx_blk = idx_ref[...]
    idx_out[...] = idx_blk
    val_out[...] = val_ref[...]
    blk_max = jnp.max(idx_blk)

    @pl.when(i == 0)
    def _init():
        acc_ref[0] = _INT_MIN

    acc_ref[0] = jnp.maximum(acc_ref[0], blk_max)

    @pl.when(i == _GRID - 1)
    def _fin():
        n = acc_ref[0] + 1
        shape_out[0] = n
        shape_out[1] = n


def kernel(indices, values):
    idx2 = indices.reshape(_GRID, _IDX_ROWS, 128)
    val2 = values.reshape(_GRID, _VAL_ROWS, 128)
    idx_out, val_out, dense_shape = pl.pallas_call(
        _fused_body,
        grid=(_GRID,),
        in_specs=[
            pl.BlockSpec((1, _IDX_ROWS, 128), lambda i: (i, 0, 0)),
            pl.BlockSpec((1, _VAL_ROWS, 128), lambda i: (i, 0, 0)),
        ],
        out_specs=[
            pl.BlockSpec((1, _IDX_ROWS, 128), lambda i: (i, 0, 0)),
            pl.BlockSpec((1, _VAL_ROWS, 128), lambda i: (i, 0, 0)),
            pl.BlockSpec(memory_space=pltpu.SMEM),
        ],
        out_shape=[
            jax.ShapeDtypeStruct((_GRID, _IDX_ROWS, 128), jnp.int32),
            jax.ShapeDtypeStruct((_GRID, _VAL_ROWS, 128), jnp.float32),
            jax.ShapeDtypeStruct((2,), jnp.int32),
        ],
        scratch_shapes=[pltpu.SMEM((1,), jnp.int32)],
    )(idx2, val2)
    return (idx_out.reshape(_E, 2), val_out.reshape(_E), dense_shape)


# speedup vs baseline: 1.2731x; 1.2731x over previous
import jax
import jax.numpy as jnp
from jax.experimental import pallas as pl
from jax.experimental.pallas import tpu as pltpu

_E = 1600000
_GRID = 25
_IDX_CH = (_E * 2) // _GRID
_VAL_CH = _E // _GRID

_INT_MIN = jnp.iinfo(jnp.int32).min


def _fused_body(idx_ref, val_ref, idx_out, val_out, shape_out, acc_ref):
    i = pl.program_id(0)
    idx_blk = idx_ref[...]
    idx_out[...] = idx_blk

    @pl.when(i == 0)
    def _copy_values():
        val_out[...] = val_ref[...]

    blk_max = jnp.max(idx_blk)

    @pl.when(i == 0)
    def _init():
        acc_ref[0] = _INT_MIN

    acc_ref[0] = jnp.maximum(acc_ref[0], blk_max)

    @pl.when(i == _GRID - 1)
    def _fin():
        n = acc_ref[0] + 1
        shape_out[0] = n
        shape_out[1] = n


def kernel(indices, values):
    idx_flat = indices.reshape(_E * 2)
    idx_out, val_out, dense_shape = pl.pallas_call(
        _fused_body,
        grid=(_GRID,),
        in_specs=[
            pl.BlockSpec((_IDX_CH,), lambda i: (i,)),
            pl.BlockSpec((_E,), lambda i: (0,)),
        ],
        out_specs=[
            pl.BlockSpec((_IDX_CH,), lambda i: (i,)),
            pl.BlockSpec((_E,), lambda i: (0,)),
            pl.BlockSpec(memory_space=pltpu.MemorySpace.SMEM),
        ],
        out_shape=[
            jax.ShapeDtypeStruct((_E * 2,), jnp.int32),
            jax.ShapeDtypeStruct((_E,), jnp.float32),
            jax.ShapeDtypeStruct((2,), jnp.int32),
        ],
        scratch_shapes=[pltpu.SMEM((1,), jnp.int32)],
    )(idx_flat, values)
    return (idx_out.reshape(_E, 2), val_out, dense_shape)
